# Initial kernel scaffold; baseline (speedup 1.0000x reference)
import jax
import jax.numpy as jnp
from jax.experimental import pallas as pl


def kernel(x, W1, b1, Ws, bs, ln_g, ln_b, Wh1, bh1, Wh2, bh2):
    raise NotImplementedError("write your pallas kernel here")


# speedup vs baseline: 2.5367x; 2.5367x over previous
import functools

import jax
import jax.numpy as jnp
from jax import lax
from jax.experimental import pallas as pl
from jax.experimental.pallas import tpu as pltpu

B, L, D, P = 64, 32, 4096, 64
K = 512
DBLK = 512
BBLK = 8


def _scores_body(v_ref, x_ref, s_ref):
    acc = x_ref[:, 0, :] * v_ref[0]
    for l in range(1, L):
        acc += x_ref[:, l, :] * v_ref[l]
    s_ref[...] = acc


def _topk_weights_body(s_ref, wd_ref):
    s = s_ref[...]
    bits = lax.bitcast_convert_type(s, jnp.int32)
    key = bits ^ (lax.shift_right_arithmetic(bits, 31) & jnp.int32(0x7FFFFFFF))
    rowmax = jnp.max(s, axis=1, keepdims=True)
    lo = jnp.min(key, axis=1, keepdims=True)
    hi = jnp.max(key, axis=1, keepdims=True) + 1

    def body(_, lohi):
        lo, hi = lohi
        mid = (lo >> 1) + (hi >> 1) + (lo & hi & 1)
        cnt = jnp.sum((key >= mid).astype(jnp.int32), axis=1, keepdims=True)
        pred = cnt >= K
        return jnp.where(pred, mid, lo), jnp.where(pred, hi, mid)

    lo, hi = lax.fori_loop(0, 32, body, (lo, hi))
    t = lo
    w = jnp.where(key >= t, jnp.exp(s - rowmax), 0.0)
    z = jnp.sum(w, axis=1, keepdims=True)
    wd_ref[...] = w / z


def _weighted_reduce_body(x_ref, wd_ref, r_ref):
    j = pl.program_id(1)

    @pl.when(j == 0)
    def _():
        r_ref[...] = jnp.zeros_like(r_ref)

    r_ref[...] += jnp.sum(x_ref[...] * wd_ref[...][:, None, :], axis=2)


def _head_body(r_ref, w1_ref, b1_ref, g_ref, bb_ref, wh1_ref, bh1_ref,
               wh2_ref, bh2_ref, out_ref):
    r = r_ref[...]
    out = lax.dot_general(r, w1_ref[...], (((1,), (1,)), ((), ())),
                          precision=lax.Precision.HIGHEST,
                          preferred_element_type=jnp.float32) + b1_ref[...]
    mu = jnp.mean(out, axis=1, keepdims=True)
    dlt = out - mu
    var = jnp.mean(dlt * dlt, axis=1, keepdims=True)
    outn = dlt * lax.rsqrt(var + 1e-5) * g_ref[...] + bb_ref[...]
    h1 = lax.dot_general(outn, wh1_ref[...], (((1,), (1,)), ((), ())),
                         precision=lax.Precision.HIGHEST,
                         preferred_element_type=jnp.float32) + bh1_ref[...]
    h1 = 0.5 * h1 * (1.0 + lax.erf(h1 * (2.0 ** -0.5)))
    out_ref[...] = lax.dot_general(h1, wh2_ref[...], (((1,), (1,)), ((), ())),
                                   precision=lax.Precision.HIGHEST,
                                   preferred_element_type=jnp.float32) + bh2_ref[...]


@jax.jit
def kernel(x, W1, b1, Ws, bs, ln_g, ln_b, Wh1, bh1, Wh2, bh2):
    v = jnp.einsum("pl,p->l", W1, Ws[0])

    scores = pl.pallas_call(
        _scores_body,
        grid=(B // BBLK, D // DBLK),
        in_specs=[
            pl.BlockSpec(memory_space=pltpu.SMEM),
            pl.BlockSpec((BBLK, L, DBLK), lambda i, j: (i, 0, j)),
        ],
        out_specs=pl.BlockSpec((BBLK, DBLK), lambda i, j: (i, j)),
        out_shape=jax.ShapeDtypeStruct((B, D), jnp.float32),
    )(v, x)

    wd = pl.pallas_call(
        _topk_weights_body,
        out_shape=jax.ShapeDtypeStruct((B, D), jnp.float32),
    )(scores)

    r = pl.pallas_call(
        _weighted_reduce_body,
        grid=(B // BBLK, D // DBLK),
        in_specs=[
            pl.BlockSpec((BBLK, L, DBLK), lambda i, j: (i, 0, j)),
            pl.BlockSpec((BBLK, DBLK), lambda i, j: (i, j)),
        ],
        out_specs=pl.BlockSpec((BBLK, L), lambda i, j: (i, 0)),
        out_shape=jax.ShapeDtypeStruct((B, L), jnp.float32),
    )(x, wd)

    logits = pl.pallas_call(
        _head_body,
        out_shape=jax.ShapeDtypeStruct((B, 2), jnp.float32),
    )(r, W1, b1.reshape(1, P), ln_g.reshape(1, P), ln_b.reshape(1, P),
      Wh1, bh1.reshape(1, 128), Wh2, bh2.reshape(1, 2))
    return logits
